# baseline (speedup 1.0000x reference)
import functools

import jax
import jax.numpy as jnp
from jax import lax
from jax.experimental import pallas as pl
from jax.experimental.pallas import tpu as pltpu
from jax.experimental.pallas import tpu_sc as plsc

S_MIN = 0.5
S_MAX = 2.0
K = 16
KRAW = 32
EPS = 1e-6
EPS2 = 1e-12

NC = 2
NS = 16
NW = NC * NS
L = 16

_MAGIC = 0x5F3759DF
_BIG = 3.0e38

_C = 14
_CR = _C * L
_CB = _C * L * K
_CB2 = _C * L * KRAW


def _sqrt16(x):
    i = plsc.bitcast(x, jnp.int32)
    i = _MAGIC - jnp.right_shift(i, 1)
    y = plsc.bitcast(i, jnp.float32)
    y = y * (1.5 - (0.5 * x) * y * y)
    return x * y


def _phase_a_body(n, wpr, ngroups,
                  px_hbm, py_hbm, pz_hbm, nb_hbm,
                  means_hbm, part_hbm,
                  xtab_sh, ytab_sh, ztab_sh,
                  own_x, own_y, own_z, means_v, part_v,
                  raw0, raw1, idx0, idx1,
                  gx0, gx1, gy0, gy1, gz0, gz1, sem0, sem1):
    wid = lax.axis_index("s") * NC + lax.axis_index("c")
    base = jnp.minimum(wid * wpr, n - wpr)
    nc = ngroups // _C

    raws = (raw0, raw1)
    idxs = (idx0, idx1)
    gxs = (gx0, gx1)
    gys = (gy0, gy1)
    gzs = (gz0, gz1)
    sems = (sem0, sem1)

    pltpu.sync_copy(px_hbm.at[pl.ds(base, wpr)], own_x)
    pltpu.sync_copy(py_hbm.at[pl.ds(base, wpr)], own_y)
    pltpu.sync_copy(pz_hbm.at[pl.ds(base, wpr)], own_z)

    @pl.when(lax.axis_index("s") == 0)
    def _():
        pltpu.sync_copy(px_hbm, xtab_sh)
        pltpu.sync_copy(py_hbm, ytab_sh)
        pltpu.sync_copy(pz_hbm, ztab_sh)

    plsc.subcore_barrier()

    zero = jnp.zeros((L,), jnp.float32)
    lane = lax.iota(jnp.int32, L)

    def fire(ci, b):
        pltpu.sync_copy(nb_hbm.at[pl.ds(base + ci * _CR, _CR), :], raws[b])

        def cbody(g, carry):
            rows = lane + g * L
            for j in range(K):
                iv = plsc.load_gather(raws[b], [rows, jnp.full((L,), j, jnp.int32)])
                idxs[b][pl.ds(g * (L * K) + j * L, L)] = iv
            return carry

        lax.fori_loop(0, _C, cbody, 0)
        pltpu.async_copy(xtab_sh.at[idxs[b]], gxs[b], sems[b])
        pltpu.async_copy(ytab_sh.at[idxs[b]], gys[b], sems[b])
        pltpu.async_copy(ztab_sh.at[idxs[b]], gzs[b], sems[b])

    fire(0, 0)
    fire(1, 1)

    def pair_body(i, carry):
        for b in range(2):
            ci = 2 * i + b
            pltpu.make_async_copy(px_hbm.at[pl.ds(0, _CB)], gxs[b], sems[b]).wait()
            pltpu.make_async_copy(py_hbm.at[pl.ds(0, _CB)], gys[b], sems[b]).wait()
            pltpu.make_async_copy(pz_hbm.at[pl.ds(0, _CB)], gzs[b], sems[b]).wait()

            def gbody(g, car):
                asum, acnt, amin, amax = car
                rb = (ci * _C + g) * L
                o = g * (L * K)
                px = own_x[pl.ds(rb, L)]
                py = own_y[pl.ds(rb, L)]
                pz = own_z[pl.ds(rb, L)]
                dsum = zero
                dcnt = zero
                for j in range(K):
                    dx = gxs[b][pl.ds(o + j * L, L)] - px
                    dy = gys[b][pl.ds(o + j * L, L)] - py
                    dz = gzs[b][pl.ds(o + j * L, L)] - pz
                    d2 = dx * dx + dy * dy + dz * dz
                    dist = _sqrt16(d2)
                    m = d2 > EPS2
                    dsum = dsum + jnp.where(m, dist, 0.0)
                    dcnt = dcnt + jnp.where(m, 1.0, 0.0)
                mean = dsum / jnp.maximum(dcnt, 1.0)
                means_v[pl.ds(rb, L)] = mean
                valid = base + rb >= wid * wpr
                has = jnp.logical_and(dcnt > 0.0, valid)
                asum = asum + jnp.where(has, mean, 0.0)
                acnt = acnt + jnp.where(has, 1.0, 0.0)
                amin = jnp.minimum(amin, jnp.where(has, mean, _BIG))
                amax = jnp.maximum(amax, jnp.where(has, mean, -_BIG))
                return asum, acnt, amin, amax

            carry = lax.fori_loop(0, _C, gbody, carry)
            nci = ci + 2

            @pl.when(nci < nc)
            def _():
                fire(nci, b)
        return carry

    init = (zero, zero, jnp.full((L,), _BIG, jnp.float32),
            jnp.full((L,), -_BIG, jnp.float32))
    asum, acnt, amin, amax = lax.fori_loop(0, nc // 2, pair_body, init)

    part_v[pl.ds(0, L)] = asum
    part_v[pl.ds(L, L)] = acnt
    part_v[pl.ds(2 * L, L)] = amin
    part_v[pl.ds(3 * L, L)] = amax
    pltpu.sync_copy(part_v, part_hbm.at[pl.ds(wid * 4 * L, 4 * L)])
    pltpu.sync_copy(means_v, means_hbm.at[pl.ds(base, wpr)])


def _phase_b_body(n, wpr, ngroups,
                  means_hbm, part_hbm, out_hbm,
                  means_v, out_v, part_v):
    wid = lax.axis_index("s") * NC + lax.axis_index("c")
    base = jnp.minimum(wid * wpr, n - wpr)

    pltpu.sync_copy(part_hbm, part_v)

    s = jnp.zeros((L,), jnp.float32)
    cnt = jnp.zeros((L,), jnp.float32)
    mn = jnp.full((L,), _BIG, jnp.float32)
    mx = jnp.full((L,), -_BIG, jnp.float32)
    for w in range(NW):
        o = w * 4 * L
        s = s + part_v[pl.ds(o, L)]
        cnt = cnt + part_v[pl.ds(o + L, L)]
        mn = jnp.minimum(mn, part_v[pl.ds(o + 2 * L, L)])
        mx = jnp.maximum(mx, part_v[pl.ds(o + 3 * L, L)])
    ssum = jnp.full((L,), jnp.sum(s), jnp.float32)
    nhas = jnp.full((L,), jnp.sum(cnt), jnp.float32)
    mmin = jnp.full((L,), jnp.min(mn), jnp.float32)
    mmax = jnp.full((L,), jnp.max(mx), jnp.float32)
    hasany = nhas > 0.0
    fb = jnp.where(hasany, ssum / jnp.maximum(nhas, 1.0), 1.0)
    mmin = jnp.where(hasany, mmin, 1.0)
    mmax = jnp.where(hasany, mmax, 1.0)
    rho_min = 1.0 / (mmax + EPS)
    rho_max = 1.0 / (mmin + EPS)
    inv_den = 1.0 / (rho_max - rho_min + EPS)

    pltpu.sync_copy(means_hbm.at[pl.ds(base, wpr)], means_v)

    def gbody(g, carry):
        rb = g * L
        mean = means_v[pl.ds(rb, L)]
        meanp = jnp.where(mean > 0.0, mean, fb)
        rho = 1.0 / (meanp + EPS)
        rn = (rho - rho_min) * inv_den
        sc = S_MIN + (S_MAX - S_MIN) * (1.0 - rn)
        sc = jnp.maximum(jnp.minimum(sc, S_MAX), S_MIN)
        out_v[pl.ds(rb, L)] = sc
        return carry

    lax.fori_loop(0, ngroups, gbody, 0)

    pltpu.sync_copy(out_v, out_hbm.at[pl.ds(base, wpr)])


@jax.jit
def kernel(points, neighbors):
    n = points.shape[0]
    wpr = -(-n // (NW * L)) * L
    ngroups = wpr // L

    p32 = points.astype(jnp.float32)
    px_h = p32[:, 0]
    py_h = p32[:, 1]
    pz_h = p32[:, 2]
    nbf = neighbors[:, :K].astype(jnp.int32)

    mesh = plsc.VectorSubcoreMesh(core_axis_name="c", subcore_axis_name="s")

    phase_a = pl.kernel(
        functools.partial(_phase_a_body, n, wpr, ngroups),
        out_type=(
            jax.ShapeDtypeStruct((n,), jnp.float32),
            jax.ShapeDtypeStruct((NW * 4 * L,), jnp.float32),
        ),
        mesh=mesh,
        scratch_types=[
            pltpu.VMEM_SHARED((n,), jnp.float32),
            pltpu.VMEM_SHARED((n,), jnp.float32),
            pltpu.VMEM_SHARED((n,), jnp.float32),
            pltpu.VMEM((wpr,), jnp.float32),
            pltpu.VMEM((wpr,), jnp.float32),
            pltpu.VMEM((wpr,), jnp.float32),
            pltpu.VMEM((wpr,), jnp.float32),
            pltpu.VMEM((4 * L,), jnp.float32),
            pltpu.VMEM((_CR, K), jnp.int32),
            pltpu.VMEM((_CR, K), jnp.int32),
            pltpu.VMEM((_CB,), jnp.int32),
            pltpu.VMEM((_CB,), jnp.int32),
            pltpu.VMEM((_CB,), jnp.float32),
            pltpu.VMEM((_CB,), jnp.float32),
            pltpu.VMEM((_CB,), jnp.float32),
            pltpu.VMEM((_CB,), jnp.float32),
            pltpu.VMEM((_CB,), jnp.float32),
            pltpu.VMEM((_CB,), jnp.float32),
            pltpu.SemaphoreType.DMA,
            pltpu.SemaphoreType.DMA,
        ],
        compiler_params=pltpu.CompilerParams(needs_layout_passes=False),
    )
    means, part = phase_a(px_h, py_h, pz_h, nbf)

    phase_b = pl.kernel(
        functools.partial(_phase_b_body, n, wpr, ngroups),
        out_type=jax.ShapeDtypeStruct((n,), jnp.float32),
        mesh=mesh,
        scratch_types=[
            pltpu.VMEM((wpr,), jnp.float32),
            pltpu.VMEM((wpr,), jnp.float32),
            pltpu.VMEM((NW * 4 * L,), jnp.float32),
        ],
        compiler_params=pltpu.CompilerParams(needs_layout_passes=False),
    )
    out = phase_b(means, part)
    return out.reshape(n, 1)


# speedup vs baseline: 1.0195x; 1.0195x over previous
import functools

import jax
import jax.numpy as jnp
from jax import lax
from jax.experimental import pallas as pl
from jax.experimental.pallas import tpu as pltpu
from jax.experimental.pallas import tpu_sc as plsc

S_MIN = 0.5
S_MAX = 2.0
K = 16
KRAW = 32
EPS = 1e-6
EPS2 = 1e-12

NC = 2
NS = 16
NW = NC * NS
L = 16

_MAGIC = 0x5F3759DF
_BIG = 3.0e38

_C = 14
_CR = _C * L
_CB = _C * L * K
_CB2 = _C * L * KRAW


def _sqrt16(x):
    i = plsc.bitcast(x, jnp.int32)
    i = _MAGIC - jnp.right_shift(i, 1)
    y = plsc.bitcast(i, jnp.float32)
    y = y * (1.5 - (0.5 * x) * y * y)
    return x * y


def _phase_a_body(n, wpr, ngroups,
                  px_hbm, py_hbm, pz_hbm, nb_hbm,
                  means_hbm, part_hbm,
                  xtab_sh, ytab_sh, ztab_sh,
                  own_x, own_y, own_z, means_v, part_v,
                  raw0, raw1, idx0, idx1,
                  gx0, gx1, gy0, gy1, gz0, gz1, sem0, sem1):
    wid = lax.axis_index("s") * NC + lax.axis_index("c")
    base = jnp.minimum(wid * wpr, n - wpr)
    nc = ngroups // _C

    raws = (raw0, raw1)
    idxs = (idx0, idx1)
    gxs = (gx0, gx1)
    gys = (gy0, gy1)
    gzs = (gz0, gz1)
    sems = (sem0, sem1)

    pltpu.sync_copy(px_hbm.at[pl.ds(base, wpr)], own_x)
    pltpu.sync_copy(py_hbm.at[pl.ds(base, wpr)], own_y)
    pltpu.sync_copy(pz_hbm.at[pl.ds(base, wpr)], own_z)

    @pl.when(lax.axis_index("s") == 0)
    def _():
        pltpu.sync_copy(px_hbm, xtab_sh)
        pltpu.sync_copy(py_hbm, ytab_sh)
        pltpu.sync_copy(pz_hbm, ztab_sh)

    plsc.subcore_barrier()

    zero = jnp.zeros((L,), jnp.float32)
    lane = lax.iota(jnp.int32, L)

    def fire(ci, b):
        pltpu.sync_copy(nb_hbm.at[pl.ds(base + ci * _CR, _CR), :], raws[b])

        def cbody(g, carry):
            rows = lane + g * L
            for j in range(K):
                iv = plsc.load_gather(raws[b], [rows, jnp.full((L,), j, jnp.int32)])
                idxs[b][pl.ds(g * (L * K) + j * L, L)] = iv
            return carry

        lax.fori_loop(0, _C, cbody, 0)
        pltpu.async_copy(xtab_sh.at[idxs[b]], gxs[b], sems[b])
        pltpu.async_copy(ytab_sh.at[idxs[b]], gys[b], sems[b])
        pltpu.async_copy(ztab_sh.at[idxs[b]], gzs[b], sems[b])

    fire(0, 0)
    fire(1, 1)

    def pair_body(i, carry):
        for b in range(2):
            ci = 2 * i + b
            pltpu.make_async_copy(px_hbm.at[pl.ds(0, _CB)], gxs[b], sems[b]).wait()
            pltpu.make_async_copy(py_hbm.at[pl.ds(0, _CB)], gys[b], sems[b]).wait()
            pltpu.make_async_copy(pz_hbm.at[pl.ds(0, _CB)], gzs[b], sems[b]).wait()

            def gbody(g, car):
                asum, acnt, amin, amax = car
                rb = (ci * _C + g) * L
                o = g * (L * K)
                px = own_x[pl.ds(rb, L)]
                py = own_y[pl.ds(rb, L)]
                pz = own_z[pl.ds(rb, L)]
                dsum = zero
                dcnt = zero
                for j in range(K):
                    dx = gxs[b][pl.ds(o + j * L, L)] - px
                    dy = gys[b][pl.ds(o + j * L, L)] - py
                    dz = gzs[b][pl.ds(o + j * L, L)] - pz
                    d2 = dx * dx + dy * dy + dz * dz
                    dist = _sqrt16(d2)
                    m = d2 > EPS2
                    dsum = dsum + jnp.where(m, dist, 0.0)
                    dcnt = dcnt + jnp.where(m, 1.0, 0.0)
                mean = dsum / jnp.maximum(dcnt, 1.0)
                means_v[pl.ds(rb, L)] = mean
                valid = base + rb >= wid * wpr
                has = jnp.logical_and(dcnt > 0.0, valid)
                asum = asum + jnp.where(has, mean, 0.0)
                acnt = acnt + jnp.where(has, 1.0, 0.0)
                amin = jnp.minimum(amin, jnp.where(has, mean, _BIG))
                amax = jnp.maximum(amax, jnp.where(has, mean, -_BIG))
                return asum, acnt, amin, amax

            carry = lax.fori_loop(0, _C, gbody, carry)
            nci = ci + 2

            @pl.when(nci < nc)
            def _():
                fire(nci, b)
        return carry

    init = (zero, zero, jnp.full((L,), _BIG, jnp.float32),
            jnp.full((L,), -_BIG, jnp.float32))
    asum, acnt, amin, amax = lax.fori_loop(0, nc // 2, pair_body, init)

    part_v[pl.ds(0, L)] = asum
    part_v[pl.ds(L, L)] = acnt
    part_v[pl.ds(2 * L, L)] = amin
    part_v[pl.ds(3 * L, L)] = amax
    pltpu.sync_copy(part_v, part_hbm.at[pl.ds(wid * 4 * L, 4 * L)])
    pltpu.sync_copy(means_v, means_hbm.at[pl.ds(base, wpr)])


def _phase_b_body(n, wpr, ngroups,
                  means_hbm, part_hbm, out_hbm,
                  means_v, out_v, part_v):
    wid = lax.axis_index("s") * NC + lax.axis_index("c")
    base = jnp.minimum(wid * wpr, n - wpr)

    pltpu.sync_copy(part_hbm, part_v)

    s = jnp.zeros((L,), jnp.float32)
    cnt = jnp.zeros((L,), jnp.float32)
    mn = jnp.full((L,), _BIG, jnp.float32)
    mx = jnp.full((L,), -_BIG, jnp.float32)
    for w in range(NW):
        o = w * 4 * L
        s = s + part_v[pl.ds(o, L)]
        cnt = cnt + part_v[pl.ds(o + L, L)]
        mn = jnp.minimum(mn, part_v[pl.ds(o + 2 * L, L)])
        mx = jnp.maximum(mx, part_v[pl.ds(o + 3 * L, L)])
    ssum = jnp.full((L,), jnp.sum(s), jnp.float32)
    nhas = jnp.full((L,), jnp.sum(cnt), jnp.float32)
    mmin = jnp.full((L,), jnp.min(mn), jnp.float32)
    mmax = jnp.full((L,), jnp.max(mx), jnp.float32)
    hasany = nhas > 0.0
    fb = jnp.where(hasany, ssum / jnp.maximum(nhas, 1.0), 1.0)
    mmin = jnp.where(hasany, mmin, 1.0)
    mmax = jnp.where(hasany, mmax, 1.0)
    rho_min = 1.0 / (mmax + EPS)
    rho_max = 1.0 / (mmin + EPS)
    inv_den = 1.0 / (rho_max - rho_min + EPS)

    pltpu.sync_copy(means_hbm.at[pl.ds(base, wpr)], means_v)

    def gbody(g, carry):
        rb = g * L
        mean = means_v[pl.ds(rb, L)]
        meanp = jnp.where(mean > 0.0, mean, fb)
        rho = 1.0 / (meanp + EPS)
        rn = (rho - rho_min) * inv_den
        sc = S_MIN + (S_MAX - S_MIN) * (1.0 - rn)
        sc = jnp.maximum(jnp.minimum(sc, S_MAX), S_MIN)
        out_v[pl.ds(rb, L)] = sc
        return carry

    lax.fori_loop(0, ngroups, gbody, 0)

    pltpu.sync_copy(out_v, out_hbm.at[pl.ds(base, wpr)])


@jax.jit
def kernel(points, neighbors):
    n = points.shape[0]
    wpr = -(-n // (NW * L)) * L
    ngroups = wpr // L

    p32 = points.astype(jnp.float32)
    px_h = p32[:, 0]
    py_h = p32[:, 1]
    pz_h = p32[:, 2]
    nbf = neighbors.astype(jnp.int32)

    mesh = plsc.VectorSubcoreMesh(core_axis_name="c", subcore_axis_name="s")

    phase_a = pl.kernel(
        functools.partial(_phase_a_body, n, wpr, ngroups),
        out_type=(
            jax.ShapeDtypeStruct((n,), jnp.float32),
            jax.ShapeDtypeStruct((NW * 4 * L,), jnp.float32),
        ),
        mesh=mesh,
        scratch_types=[
            pltpu.VMEM_SHARED((n,), jnp.float32),
            pltpu.VMEM_SHARED((n,), jnp.float32),
            pltpu.VMEM_SHARED((n,), jnp.float32),
            pltpu.VMEM((wpr,), jnp.float32),
            pltpu.VMEM((wpr,), jnp.float32),
            pltpu.VMEM((wpr,), jnp.float32),
            pltpu.VMEM((wpr,), jnp.float32),
            pltpu.VMEM((4 * L,), jnp.float32),
            pltpu.VMEM((_CR, KRAW), jnp.int32),
            pltpu.VMEM((_CR, KRAW), jnp.int32),
            pltpu.VMEM((_CB,), jnp.int32),
            pltpu.VMEM((_CB,), jnp.int32),
            pltpu.VMEM((_CB,), jnp.float32),
            pltpu.VMEM((_CB,), jnp.float32),
            pltpu.VMEM((_CB,), jnp.float32),
            pltpu.VMEM((_CB,), jnp.float32),
            pltpu.VMEM((_CB,), jnp.float32),
            pltpu.VMEM((_CB,), jnp.float32),
            pltpu.SemaphoreType.DMA,
            pltpu.SemaphoreType.DMA,
        ],
        compiler_params=pltpu.CompilerParams(needs_layout_passes=False),
    )
    means, part = phase_a(px_h, py_h, pz_h, nbf)

    phase_b = pl.kernel(
        functools.partial(_phase_b_body, n, wpr, ngroups),
        out_type=jax.ShapeDtypeStruct((n,), jnp.float32),
        mesh=mesh,
        scratch_types=[
            pltpu.VMEM((wpr,), jnp.float32),
            pltpu.VMEM((wpr,), jnp.float32),
            pltpu.VMEM((NW * 4 * L,), jnp.float32),
        ],
        compiler_params=pltpu.CompilerParams(needs_layout_passes=False),
    )
    out = phase_b(means, part)
    return out.reshape(n, 1)


# speedup vs baseline: 1.0216x; 1.0020x over previous
import functools

import jax
import jax.numpy as jnp
from jax import lax
from jax.experimental import pallas as pl
from jax.experimental.pallas import tpu as pltpu
from jax.experimental.pallas import tpu_sc as plsc

S_MIN = 0.5
S_MAX = 2.0
K = 16
KRAW = 32
EPS = 1e-6
EPS2 = 1e-12

NC = 2
NS = 16
NW = NC * NS
L = 16

_MAGIC = 0x5F3759DF
_BIG = 3.0e38

_C = 14
_CR = _C * L
_CB = _C * L * K
_CB2 = _C * L * KRAW


def _sqrt16(x):
    i = plsc.bitcast(x, jnp.int32)
    i = _MAGIC - jnp.right_shift(i, 1)
    y = plsc.bitcast(i, jnp.float32)
    hx = 0.5 * x
    y = y * (1.5 - hx * y * y)
    y = y * (1.5 - hx * y * y)
    return x * y


def _phase_a_body(n, wpr, ngroups,
                  px_hbm, py_hbm, pz_hbm, nb_hbm,
                  means_hbm, part_hbm,
                  xtab_sh, ytab_sh, ztab_sh,
                  own_x, own_y, own_z, means_v, part_v,
                  raw0, raw1, idx0, idx1,
                  gx0, gx1, gy0, gy1, gz0, gz1, sem0, sem1):
    wid = lax.axis_index("s") * NC + lax.axis_index("c")
    base = jnp.minimum(wid * wpr, n - wpr)
    nc = ngroups // _C

    raws = (raw0, raw1)
    idxs = (idx0, idx1)
    gxs = (gx0, gx1)
    gys = (gy0, gy1)
    gzs = (gz0, gz1)
    sems = (sem0, sem1)

    pltpu.sync_copy(px_hbm.at[pl.ds(base, wpr)], own_x)
    pltpu.sync_copy(py_hbm.at[pl.ds(base, wpr)], own_y)
    pltpu.sync_copy(pz_hbm.at[pl.ds(base, wpr)], own_z)

    @pl.when(lax.axis_index("s") == 0)
    def _():
        pltpu.sync_copy(px_hbm, xtab_sh)
        pltpu.sync_copy(py_hbm, ytab_sh)
        pltpu.sync_copy(pz_hbm, ztab_sh)

    plsc.subcore_barrier()

    zero = jnp.zeros((L,), jnp.float32)
    lane = lax.iota(jnp.int32, L)

    def fire(ci, b):
        pltpu.sync_copy(nb_hbm.at[pl.ds(base + ci * _CR, _CR), :], raws[b])

        def cbody(g, carry):
            rows = lane + g * L
            for j in range(K):
                iv = plsc.load_gather(raws[b], [rows, jnp.full((L,), j, jnp.int32)])
                idxs[b][pl.ds(g * (L * K) + j * L, L)] = iv
            return carry

        lax.fori_loop(0, _C, cbody, 0)
        pltpu.async_copy(xtab_sh.at[idxs[b]], gxs[b], sems[b])
        pltpu.async_copy(ytab_sh.at[idxs[b]], gys[b], sems[b])
        pltpu.async_copy(ztab_sh.at[idxs[b]], gzs[b], sems[b])

    fire(0, 0)
    fire(1, 1)

    def pair_body(i, carry):
        for b in range(2):
            ci = 2 * i + b
            pltpu.make_async_copy(px_hbm.at[pl.ds(0, _CB)], gxs[b], sems[b]).wait()
            pltpu.make_async_copy(py_hbm.at[pl.ds(0, _CB)], gys[b], sems[b]).wait()
            pltpu.make_async_copy(pz_hbm.at[pl.ds(0, _CB)], gzs[b], sems[b]).wait()

            def gbody(g, car):
                asum, acnt, amin, amax = car
                rb = (ci * _C + g) * L
                o = g * (L * K)
                px = own_x[pl.ds(rb, L)]
                py = own_y[pl.ds(rb, L)]
                pz = own_z[pl.ds(rb, L)]
                dsum = zero
                dcnt = zero
                for j in range(K):
                    dx = gxs[b][pl.ds(o + j * L, L)] - px
                    dy = gys[b][pl.ds(o + j * L, L)] - py
                    dz = gzs[b][pl.ds(o + j * L, L)] - pz
                    d2 = dx * dx + dy * dy + dz * dz
                    dist = _sqrt16(d2)
                    m = d2 > EPS2
                    dsum = dsum + jnp.where(m, dist, 0.0)
                    dcnt = dcnt + jnp.where(m, 1.0, 0.0)
                mean = dsum / jnp.maximum(dcnt, 1.0)
                means_v[pl.ds(rb, L)] = mean
                valid = base + rb >= wid * wpr
                has = jnp.logical_and(dcnt > 0.0, valid)
                asum = asum + jnp.where(has, mean, 0.0)
                acnt = acnt + jnp.where(has, 1.0, 0.0)
                amin = jnp.minimum(amin, jnp.where(has, mean, _BIG))
                amax = jnp.maximum(amax, jnp.where(has, mean, -_BIG))
                return asum, acnt, amin, amax

            carry = lax.fori_loop(0, _C, gbody, carry)
            nci = ci + 2

            @pl.when(nci < nc)
            def _():
                fire(nci, b)
        return carry

    init = (zero, zero, jnp.full((L,), _BIG, jnp.float32),
            jnp.full((L,), -_BIG, jnp.float32))
    asum, acnt, amin, amax = lax.fori_loop(0, nc // 2, pair_body, init)

    part_v[pl.ds(0, L)] = asum
    part_v[pl.ds(L, L)] = acnt
    part_v[pl.ds(2 * L, L)] = amin
    part_v[pl.ds(3 * L, L)] = amax
    pltpu.sync_copy(part_v, part_hbm.at[pl.ds(wid * 4 * L, 4 * L)])
    pltpu.sync_copy(means_v, means_hbm.at[pl.ds(base, wpr)])


def _phase_b_body(n, wpr, ngroups,
                  means_hbm, part_hbm, out_hbm,
                  means_v, out_v, part_v):
    wid = lax.axis_index("s") * NC + lax.axis_index("c")
    base = jnp.minimum(wid * wpr, n - wpr)

    pltpu.sync_copy(part_hbm, part_v)

    s = jnp.zeros((L,), jnp.float32)
    cnt = jnp.zeros((L,), jnp.float32)
    mn = jnp.full((L,), _BIG, jnp.float32)
    mx = jnp.full((L,), -_BIG, jnp.float32)
    for w in range(NW):
        o = w * 4 * L
        s = s + part_v[pl.ds(o, L)]
        cnt = cnt + part_v[pl.ds(o + L, L)]
        mn = jnp.minimum(mn, part_v[pl.ds(o + 2 * L, L)])
        mx = jnp.maximum(mx, part_v[pl.ds(o + 3 * L, L)])
    ssum = jnp.full((L,), jnp.sum(s), jnp.float32)
    nhas = jnp.full((L,), jnp.sum(cnt), jnp.float32)
    mmin = jnp.full((L,), jnp.min(mn), jnp.float32)
    mmax = jnp.full((L,), jnp.max(mx), jnp.float32)
    hasany = nhas > 0.0
    fb = jnp.where(hasany, ssum / jnp.maximum(nhas, 1.0), 1.0)
    mmin = jnp.where(hasany, mmin, 1.0)
    mmax = jnp.where(hasany, mmax, 1.0)
    rho_min = 1.0 / (mmax + EPS)
    rho_max = 1.0 / (mmin + EPS)
    inv_den = 1.0 / (rho_max - rho_min + EPS)

    pltpu.sync_copy(means_hbm.at[pl.ds(base, wpr)], means_v)

    def gbody(g, carry):
        rb = g * L
        mean = means_v[pl.ds(rb, L)]
        meanp = jnp.where(mean > 0.0, mean, fb)
        rho = 1.0 / (meanp + EPS)
        rn = (rho - rho_min) * inv_den
        sc = S_MIN + (S_MAX - S_MIN) * (1.0 - rn)
        sc = jnp.maximum(jnp.minimum(sc, S_MAX), S_MIN)
        out_v[pl.ds(rb, L)] = sc
        return carry

    lax.fori_loop(0, ngroups, gbody, 0)

    pltpu.sync_copy(out_v, out_hbm.at[pl.ds(base, wpr)])


@jax.jit
def kernel(points, neighbors):
    n = points.shape[0]
    wpr = -(-n // (NW * L)) * L
    ngroups = wpr // L

    p32 = points.astype(jnp.float32)
    px_h = p32[:, 0]
    py_h = p32[:, 1]
    pz_h = p32[:, 2]
    nbf = neighbors.astype(jnp.int32)

    mesh = plsc.VectorSubcoreMesh(core_axis_name="c", subcore_axis_name="s")

    phase_a = pl.kernel(
        functools.partial(_phase_a_body, n, wpr, ngroups),
        out_type=(
            jax.ShapeDtypeStruct((n,), jnp.float32),
            jax.ShapeDtypeStruct((NW * 4 * L,), jnp.float32),
        ),
        mesh=mesh,
        scratch_types=[
            pltpu.VMEM_SHARED((n,), jnp.float32),
            pltpu.VMEM_SHARED((n,), jnp.float32),
            pltpu.VMEM_SHARED((n,), jnp.float32),
            pltpu.VMEM((wpr,), jnp.float32),
            pltpu.VMEM((wpr,), jnp.float32),
            pltpu.VMEM((wpr,), jnp.float32),
            pltpu.VMEM((wpr,), jnp.float32),
            pltpu.VMEM((4 * L,), jnp.float32),
            pltpu.VMEM((_CR, KRAW), jnp.int32),
            pltpu.VMEM((_CR, KRAW), jnp.int32),
            pltpu.VMEM((_CB,), jnp.int32),
            pltpu.VMEM((_CB,), jnp.int32),
            pltpu.VMEM((_CB,), jnp.float32),
            pltpu.VMEM((_CB,), jnp.float32),
            pltpu.VMEM((_CB,), jnp.float32),
            pltpu.VMEM((_CB,), jnp.float32),
            pltpu.VMEM((_CB,), jnp.float32),
            pltpu.VMEM((_CB,), jnp.float32),
            pltpu.SemaphoreType.DMA,
            pltpu.SemaphoreType.DMA,
        ],
        compiler_params=pltpu.CompilerParams(needs_layout_passes=False),
    )
    means, part = phase_a(px_h, py_h, pz_h, nbf)

    phase_b = pl.kernel(
        functools.partial(_phase_b_body, n, wpr, ngroups),
        out_type=jax.ShapeDtypeStruct((n,), jnp.float32),
        mesh=mesh,
        scratch_types=[
            pltpu.VMEM((wpr,), jnp.float32),
            pltpu.VMEM((wpr,), jnp.float32),
            pltpu.VMEM((NW * 4 * L,), jnp.float32),
        ],
        compiler_params=pltpu.CompilerParams(needs_layout_passes=False),
    )
    out = phase_b(means, part)
    return out.reshape(n, 1)
